# Initial kernel scaffold; baseline (speedup 1.0000x reference)
import jax
import jax.numpy as jnp
from jax.experimental import pallas as pl


def kernel(ids, table):
    raise NotImplementedError("write your pallas kernel here")


# speedup vs baseline: 8.2701x; 8.2701x over previous
import functools

import jax
import jax.numpy as jnp
from jax import lax
from jax.experimental import pallas as pl
from jax.experimental.pallas import tpu as pltpu
from jax.experimental.pallas import tpu_sc as plsc

NC = 2
NS = 16
NW = NC * NS
CHUNK = 128


@functools.partial(jax.jit, static_argnums=(2, 3))
def _gather_rows(ids_r, table, n_chunks, d):
    total = NW * n_chunks * CHUNK

    @functools.partial(
        pl.kernel,
        mesh=plsc.VectorSubcoreMesh(
            core_axis_name="c", subcore_axis_name="s",
            num_cores=NC, num_subcores=NS,
        ),
        out_type=jax.ShapeDtypeStruct((total, d), jnp.float32),
        scratch_types=[
            pltpu.VMEM((n_chunks, CHUNK), jnp.int32),
            pltpu.VMEM((CHUNK, d), jnp.float32),
            pltpu.SemaphoreType.DMA,
        ],
        compiler_params=pltpu.CompilerParams(use_tc_tiling_on_sc=False),
    )
    def body(ids_hbm, table_hbm, out_hbm, idx_v, rows_v, sem):
        wid = lax.axis_index("s") * NC + lax.axis_index("c")
        pltpu.sync_copy(ids_hbm.at[wid], idx_v)
        base = wid * (n_chunks * CHUNK)

        def step(j, carry):
            pltpu.async_copy(table_hbm.at[idx_v.at[j]], rows_v, sem).wait()
            pltpu.sync_copy(rows_v, out_hbm.at[pl.ds(base + j * CHUNK, CHUNK)])
            return carry

        lax.fori_loop(0, n_chunks, step, 0)

    return body(ids_r, table)


def kernel(ids, table):
    b, l = ids.shape
    v, d = table.shape
    total = b * l
    ids_flat = ids.reshape(-1).astype(jnp.int32)

    per_w = -(-total // NW)
    n_chunks = -(-per_w // CHUNK)
    padded = NW * n_chunks * CHUNK
    if padded != total:
        ids_flat = jnp.pad(ids_flat, (0, padded - total))
    ids_r = ids_flat.reshape(NW, n_chunks, CHUNK)

    out = _gather_rows(ids_r, table, n_chunks, d)
    return out[:total].reshape(b, l, d)


# speedup vs baseline: 9.4855x; 1.1470x over previous
import functools

import jax
import jax.numpy as jnp
from jax import lax
from jax.experimental import pallas as pl
from jax.experimental.pallas import tpu as pltpu
from jax.experimental.pallas import tpu_sc as plsc

NC = 2
NS = 16
NW = NC * NS
CHUNK = 128


@functools.partial(jax.jit, static_argnums=(2, 3))
def _gather_rows(ids_r, table, n_chunks, d):
    total = NW * n_chunks * CHUNK

    NBUF = 5
    LOOK = NBUF - 2
    assert n_chunks % NBUF == 0 and n_chunks // NBUF >= 3

    @functools.partial(
        pl.kernel,
        mesh=plsc.VectorSubcoreMesh(
            core_axis_name="c", subcore_axis_name="s",
            num_cores=NC, num_subcores=NS,
        ),
        out_type=jax.ShapeDtypeStruct((total, d), jnp.float32),
        scratch_types=[
            pltpu.VMEM((n_chunks, CHUNK), jnp.int32),
            pltpu.VMEM((NBUF, CHUNK, d), jnp.float32),
            pltpu.SemaphoreType.DMA,
            pltpu.SemaphoreType.DMA,
        ],
        compiler_params=pltpu.CompilerParams(use_tc_tiling_on_sc=False),
    )
    def body(ids_hbm, table_hbm, out_hbm, idx_v, rows_v, gsem, ssem):
        wid = lax.axis_index("s") * NC + lax.axis_index("c")
        pltpu.sync_copy(ids_hbm.at[wid], idx_v)
        base = wid * (n_chunks * CHUNK)

        def g_copy(slot, j):
            return pltpu.make_async_copy(
                table_hbm.at[idx_v.at[j]], rows_v.at[slot], gsem)

        def s_copy(slot, j):
            return pltpu.make_async_copy(
                rows_v.at[slot], out_hbm.at[pl.ds(base + j * CHUNK, CHUNK)],
                ssem)

        for b in range(LOOK):
            g_copy(b, b).start()

        def block(j0, first, last):
            for b in range(NBUF):
                j = j0 + b
                g_copy(b, j).wait()
                s_copy(b, j).start()
                if not (last and b >= NBUF - LOOK):
                    slot2 = (b + LOOK) % NBUF
                    if not (first and b < NBUF - LOOK):
                        s_copy(slot2, j - (NBUF - LOOK)).wait()
                    g_copy(slot2, j + LOOK).start()

        block(0, True, False)
        if n_chunks // NBUF > 2:
            def mid(t, carry):
                block(t * NBUF, False, False)
                return carry
            lax.fori_loop(1, n_chunks // NBUF - 1, mid, 0)
        block(n_chunks - NBUF, False, True)

        for j in range(n_chunks - NBUF, n_chunks):
            s_copy(j % NBUF, j).wait()

    return body(ids_r, table)


def kernel(ids, table):
    b, l = ids.shape
    v, d = table.shape
    total = b * l
    ids_flat = ids.reshape(-1).astype(jnp.int32)

    per_w = -(-total // NW)
    n_chunks = -(-per_w // CHUNK)
    padded = NW * n_chunks * CHUNK
    if padded != total:
        ids_flat = jnp.pad(ids_flat, (0, padded - total))
    ids_r = ids_flat.reshape(NW, n_chunks, CHUNK)

    out = _gather_rows(ids_r, table, n_chunks, d)
    return out[:total].reshape(b, l, d)
